# baseline (speedup 1.0000x reference)
import functools

import jax
import jax.numpy as jnp
from jax import lax
from jax.experimental import pallas as pl
from jax.experimental.pallas import tpu as pltpu
from jax.experimental.pallas import tpu_sc as plsc

_WEIGHT1 = 1.0
_WEIGHT2 = 0.5
_SIGMA1 = 1.0
_SIGMA2 = 2.0
_WEIGHT_MARGIN = 0.8

_NC = 2
_NS = 16
_L = 16
_GCHUNK = 128


def _sc_partials(preds_flat, targets, margin, score, B, C):
    nw = _NC * _NS
    bpw = B // nw
    nv = bpw // _L
    ng = bpw // _GCHUNK

    mesh = plsc.VectorSubcoreMesh(core_axis_name="c", subcore_axis_name="s")

    @functools.partial(
        pl.kernel,
        mesh=mesh,
        out_type=jax.ShapeDtypeStruct((nw * _L,), jnp.float32),
        scratch_types=[
            pltpu.VMEM((bpw,), jnp.int32),
            pltpu.VMEM((bpw,), jnp.float32),
            pltpu.VMEM((bpw,), jnp.float32),
            pltpu.VMEM((bpw,), jnp.float32),
            pltpu.VMEM((bpw,), jnp.int32),
            pltpu.VMEM((_L,), jnp.float32),
            pltpu.SemaphoreType.DMA,
        ],
    )
    def body(preds_hbm, tgt_hbm, mar_hbm, sco_hbm, out_hbm,
             idx_v, n_v, m_v, s_v, t_v, p_v, sem):
        wid = lax.axis_index("s") * _NC + lax.axis_index("c")
        base = wid * bpw
        lane = lax.iota(jnp.int32, _L)
        lane_c = lane * C

        pltpu.sync_copy(tgt_hbm.at[pl.ds(base, bpw)], t_v)

        def build(j, carry):
            sl = pl.ds(j * _L, _L)
            idx_v[sl] = t_v[sl] + (base + j * _L) * C + lane_c
            return carry

        lax.fori_loop(0, nv, build, 0)

        copies = [
            pltpu.async_copy(
                preds_hbm.at[idx_v.at[pl.ds(k * _GCHUNK, _GCHUNK)]],
                n_v.at[pl.ds(k * _GCHUNK, _GCHUNK)],
                sem,
            )
            for k in range(ng)
        ]
        pltpu.sync_copy(mar_hbm.at[pl.ds(base, bpw)], m_v)
        pltpu.sync_copy(sco_hbm.at[pl.ds(base, bpw)], s_v)
        for cp in copies:
            cp.wait()

        zero = jnp.zeros((_L,), jnp.float32)

        def step(j, accs):
            a1, a2, asum, acnt = accs
            sl = pl.ds(j * _L, _L)
            n = n_v[sl]
            m = m_v[sl]
            s = s_v[sl]
            m2 = m * m
            a1 = a1 + jnp.where(m > 0, _WEIGHT1 * jnp.exp(-_SIGMA1 * m2) * n, 0.0)
            a2 = a2 + jnp.where(m < 0, _WEIGHT2 * jnp.exp(-_SIGMA2 * m2) * n, 0.0)
            neg = s < 0
            asum = asum + jnp.where(neg, s, 0.0)
            acnt = acnt + jnp.where(neg, 1.0, 0.0)
            return a1, a2, asum, acnt

        a1, a2, asum, acnt = lax.fori_loop(0, nv, step, (zero, zero, zero, zero))

        def lane_total(v):
            for sh in (8, 4, 2, 1):
                v = v + v.at[lane ^ sh].get(mode="promise_in_bounds")
            return v

        p_v[...] = (jnp.where(lane == 0, lane_total(a1), 0.0)
                    + jnp.where(lane == 1, lane_total(a2), 0.0)
                    + jnp.where(lane == 2, lane_total(asum), 0.0)
                    + jnp.where(lane == 3, lane_total(acnt), 0.0))
        pltpu.sync_copy(p_v, out_hbm.at[pl.ds(wid * _L, _L)])

    return body(preds_flat, targets, margin, score)


def kernel(preds, targets, margin, score):
    B, C = preds.shape
    parts = _sc_partials(preds.reshape(B * C), targets, margin, score, B, C)
    p = parts.reshape(_NC * _NS, _L)
    loss1 = p[:, 0].sum()
    loss2 = p[:, 1].sum()
    neg_sum = p[:, 2].sum()
    neg_cnt = p[:, 3].sum()
    return -(loss1 + loss2) / B + _WEIGHT_MARGIN * (neg_sum / neg_cnt)


# speedup vs baseline: 1.4963x; 1.4963x over previous
import functools

import jax
import jax.numpy as jnp
from jax import lax
from jax.experimental import pallas as pl
from jax.experimental.pallas import tpu as pltpu

_WEIGHT1 = 1.0
_WEIGHT2 = 0.5
_SIGMA1 = 1.0
_SIGMA2 = 2.0
_WEIGHT_MARGIN = 0.8

_BR = 512


def _body(preds_ref, tgt_ref, mar_ref, sco_ref, out_ref):
    i = pl.program_id(0)

    @pl.when(i == 0)
    def _():
        out_ref[...] = jnp.zeros_like(out_ref)

    sl = pl.ds(i * _BR, _BR)
    t = tgt_ref[sl]
    m = mar_ref[sl]
    s = sco_ref[sl]

    cols = lax.broadcasted_iota(jnp.int32, preds_ref.shape, 1)
    masked = jnp.where(cols == t[:, None], preds_ref[...], 0.0)
    n = jnp.sum(masked, axis=1)

    m2 = m * m
    w = (jnp.where(m > 0, _WEIGHT1 * jnp.exp(-_SIGMA1 * m2), 0.0)
         + jnp.where(m < 0, _WEIGHT2 * jnp.exp(-_SIGMA2 * m2), 0.0))
    s_loss = jnp.sum(w * n)

    neg = s < 0
    s_neg = jnp.sum(jnp.where(neg, s, 0.0))
    c_neg = jnp.sum(jnp.where(neg, 1.0, 0.0))

    r = lax.broadcasted_iota(jnp.int32, out_ref.shape, 0)
    c = lax.broadcasted_iota(jnp.int32, out_ref.shape, 1)
    first = c == 0
    out_ref[...] += (jnp.where((r == 0) & first, s_loss, 0.0)
                     + jnp.where((r == 1) & first, s_neg, 0.0)
                     + jnp.where((r == 2) & first, c_neg, 0.0))


def kernel(preds, targets, margin, score):
    B, C = preds.shape
    grid = B // _BR
    out = pl.pallas_call(
        _body,
        grid=(grid,),
        in_specs=[
            pl.BlockSpec((_BR, C), lambda i: (i, 0)),
            pl.BlockSpec((B,), lambda i: (0,)),
            pl.BlockSpec((B,), lambda i: (0,)),
            pl.BlockSpec((B,), lambda i: (0,)),
        ],
        out_specs=pl.BlockSpec((8, 128), lambda i: (0, 0)),
        out_shape=jax.ShapeDtypeStruct((8, 128), jnp.float32),
        compiler_params=pltpu.CompilerParams(
            dimension_semantics=("arbitrary",),
        ),
    )(preds, targets, margin, score)
    return -out[0, 0] / B + _WEIGHT_MARGIN * (out[1, 0] / out[2, 0])


# speedup vs baseline: 1.5439x; 1.0318x over previous
import jax
import jax.numpy as jnp
from jax import lax
from jax.experimental import pallas as pl
from jax.experimental.pallas import tpu as pltpu

_WEIGHT1 = 1.0
_WEIGHT2 = 0.5
_SIGMA1 = 1.0
_SIGMA2 = 2.0
_WEIGHT_MARGIN = 0.8

_BR = 512


def _body(preds_ref, tgt_ref, mar_ref, sco_ref, out_ref, acc_ref, sc_ref):
    i = pl.program_id(0)
    n_steps = pl.num_programs(0)

    @pl.when(i == 0)
    def _():
        acc_ref[...] = jnp.zeros_like(acc_ref)
        sc_ref[...] = jnp.zeros_like(sc_ref)

    sl = pl.ds(i * _BR, _BR)
    t = tgt_ref[sl]
    m = mar_ref[sl]
    s = sco_ref[sl]

    m2 = m * m
    w = (jnp.where(m > 0, _WEIGHT1 * jnp.exp(-_SIGMA1 * m2), 0.0)
         + jnp.where(m < 0, _WEIGHT2 * jnp.exp(-_SIGMA2 * m2), 0.0))

    cols = lax.broadcasted_iota(jnp.int32, preds_ref.shape, 1)
    contrib = jnp.where(cols == t[:, None], w[:, None] * preds_ref[...], 0.0)
    acc_ref[...] += jnp.sum(
        contrib.reshape(_BR // 8, 8, preds_ref.shape[1]), axis=0)

    neg = s < 0
    s_neg = jnp.sum(jnp.where(neg, s, 0.0))
    c_neg = jnp.sum(jnp.where(neg, 1.0, 0.0))
    r = lax.broadcasted_iota(jnp.int32, sc_ref.shape, 0)
    c = lax.broadcasted_iota(jnp.int32, sc_ref.shape, 1)
    first = c == 0
    sc_ref[...] += (jnp.where((r == 1) & first, s_neg, 0.0)
                    + jnp.where((r == 2) & first, c_neg, 0.0))

    @pl.when(i == n_steps - 1)
    def _():
        w_loss = jnp.sum(acc_ref[...])
        out_ref[...] = sc_ref[...] + jnp.where((r == 0) & first, w_loss, 0.0)


def kernel(preds, targets, margin, score):
    B, C = preds.shape
    grid = B // _BR
    out = pl.pallas_call(
        _body,
        grid=(grid,),
        in_specs=[
            pl.BlockSpec((_BR, C), lambda i: (i, 0)),
            pl.BlockSpec((B,), lambda i: (0,)),
            pl.BlockSpec((B,), lambda i: (0,)),
            pl.BlockSpec((B,), lambda i: (0,)),
        ],
        out_specs=pl.BlockSpec((8, 128), lambda i: (0, 0)),
        out_shape=jax.ShapeDtypeStruct((8, 128), jnp.float32),
        scratch_shapes=[
            pltpu.VMEM((8, C), jnp.float32),
            pltpu.VMEM((8, 128), jnp.float32),
        ],
        compiler_params=pltpu.CompilerParams(
            dimension_semantics=("arbitrary",),
        ),
    )(preds, targets, margin, score)
    return -out[0, 0] / B + _WEIGHT_MARGIN * (out[1, 0] / out[2, 0])
